# baseline (speedup 1.0000x reference)
import functools

import jax
import jax.numpy as jnp
from jax import lax
from jax.experimental import pallas as pl
from jax.experimental.pallas import tpu as pltpu
from jax.experimental.pallas import tpu_sc as plsc

_K = 16
_EPS = 1e-8
_N = 2048
_R = 128
_SWEEPS = 6


def _jacobi3_min_eigvec(a00, a11, a22, a01, a02, a12):
    A = [[a00, a01, a02], [a01, a11, a12], [a02, a12, a22]]
    one = jnp.ones_like(a00)
    zero = jnp.zeros_like(a00)
    V = [[one, zero, zero], [zero, one, zero], [zero, zero, one]]
    for _ in range(_SWEEPS):
        for (p, q) in ((0, 2), (1, 2), (0, 1)):
            app, aqq, apq = A[p][p], A[q][q], A[p][q]
            nz = apq != 0.0
            tau = (aqq - app) / (2.0 * apq)
            sgn = jnp.where(tau >= 0.0, 1.0, -1.0)
            t = jnp.where(nz, sgn / (jnp.abs(tau) + jnp.sqrt(1.0 + tau * tau)), 0.0)
            c = jax.lax.rsqrt(1.0 + t * t)
            s = t * c
            r = 3 - p - q
            arp, arq = A[r][p], A[r][q]
            new_app = c * (c * app - s * apq) - s * (c * apq - s * aqq)
            new_aqq = s * (s * app + c * apq) + c * (s * apq + c * aqq)
            new_apq = c * s * (app - aqq) + (c * c - s * s) * apq
            new_arp = c * arp - s * arq
            new_arq = s * arp + c * arq
            A[p][p] = new_app
            A[q][q] = new_aqq
            A[p][q] = new_apq
            A[q][p] = new_apq
            A[r][p] = new_arp
            A[p][r] = new_arp
            A[r][q] = new_arq
            A[q][r] = new_arq
            for i in range(3):
                vip, viq = V[i][p], V[i][q]
                V[i][p] = c * vip - s * viq
                V[i][q] = s * vip + c * viq
    w0, w1, w2 = A[0][0], A[1][1], A[2][2]
    sel0 = (w0 <= w1) & (w0 <= w2)
    sel1 = jnp.logical_not(sel0) & (w1 <= w2)
    nx = jnp.where(sel0, V[0][0], jnp.where(sel1, V[0][1], V[0][2]))
    ny = jnp.where(sel0, V[1][0], jnp.where(sel1, V[1][1], V[1][2]))
    nz_ = jnp.where(sel0, V[2][0], jnp.where(sel1, V[2][1], V[2][2]))
    return nx, ny, nz_


def _normals_for_cloud(xt, rt):
    x, y, z = xt[0:1, :], xt[1:2, :], xt[2:3, :]
    sq_all = (x * x + y * y) + z * z
    sq_all_col = sq_all.reshape(_N, 1)
    rx, ry, rz = rt[0:1, :], rt[1:2, :], rt[2:3, :]
    sq_rows = (rx * rx + ry * ry) + rz * rz
    inner = jax.lax.dot_general(
        xt, rt, (((0,), (0,)), ((), ())),
        preferred_element_type=jnp.float32)
    dist = (sq_rows - 2.0 * inner) + sq_all_col

    iota = jax.lax.broadcasted_iota(jnp.int32, (_N, _R), 0)
    inf = jnp.float32(jnp.inf)
    for _ in range(_K):
        jmin = jnp.argmin(dist, axis=0).reshape(1, _R)
        sel = iota == jmin
        dist = jnp.where(sel, inf, dist)

    maskf = (dist == inf).astype(jnp.float32)
    mom = jax.lax.dot_general(
        xt, maskf, (((1,), (0,)), ((), ())),
        preferred_element_type=jnp.float32)
    inv_k = jnp.float32(1.0 / _K)
    mx = mom[0:1, :] * inv_k
    my = mom[1:2, :] * inv_k
    mz = mom[2:3, :] * inv_k
    xc = x.reshape(_N, 1) - mx
    yc = y.reshape(_N, 1) - my
    zc = z.reshape(_N, 1) - mz
    xm = maskf * xc
    ym = maskf * yc
    zm = maskf * zc
    ones_row = jnp.ones((1, _N), dtype=jnp.float32)

    def _colsum(v):
        return jax.lax.dot_general(
            ones_row, v, (((1,), (0,)), ((), ())),
            preferred_element_type=jnp.float32)

    cxx = _colsum(xm * xc) * inv_k
    cyy = _colsum(ym * yc) * inv_k
    czz = _colsum(zm * zc) * inv_k
    cxy = _colsum(xm * yc) * inv_k
    cxz = _colsum(xm * zc) * inv_k
    cyz = _colsum(ym * zc) * inv_k
    return _jacobi3_min_eigvec(cxx, cyy, czz, cxy, cxz, cyz)


def _fused_kernel(gt_t_ref, gt_r_ref, pr_t_ref, pr_r_ref, out_ref):
    gx, gy, gz = _normals_for_cloud(gt_t_ref[0], gt_r_ref[0])
    px, py, pz = _normals_for_cloud(pr_t_ref[0], pr_r_ref[0])
    num = gx * px + gy * py + gz * pz
    na = jnp.maximum(jnp.sqrt(gx * gx + gy * gy + gz * gz), _EPS)
    nb = jnp.maximum(jnp.sqrt(px * px + py * py + pz * pz), _EPS)
    loss = 1.0 - num / (na * nb)
    s = jnp.sum(loss)
    out_ref[...] = jnp.broadcast_to(s.reshape(1, 1, 1, 1), (1, 1, 1, 128))


def _sc_gather_rows(table, idx):
    M, D = table.shape
    B_ = idx.shape[0]
    info = plsc.get_sparse_core_info()
    nw = info.num_cores * info.num_subcores
    b_per_w = B_ // nw
    mesh = plsc.VectorSubcoreMesh(core_axis_name="c", subcore_axis_name="s")

    @functools.partial(
        pl.kernel, mesh=mesh,
        out_type=jax.ShapeDtypeStruct((B_, D), jnp.float32),
        scratch_types=[
            pltpu.VMEM((b_per_w,), jnp.int32),
            pltpu.VMEM((b_per_w, D), jnp.float32),
            pltpu.SemaphoreType.DMA,
        ],
        compiler_params=pltpu.CompilerParams(use_tc_tiling_on_sc=False),
    )
    def gather_kernel(table_hbm, idx_hbm, out_hbm, idx_v, rows_v, sem):
        wid = lax.axis_index("s") * info.num_cores + lax.axis_index("c")
        base = wid * b_per_w
        pltpu.sync_copy(idx_hbm.at[pl.ds(base, b_per_w)], idx_v)
        pltpu.async_copy(table_hbm.at[idx_v], rows_v, sem).wait()
        pltpu.sync_copy(rows_v, out_hbm.at[pl.ds(base, b_per_w)])

    return gather_kernel(table, idx)


@jax.jit
def kernel(gt, pred, idx12):
    B, N, _ = gt.shape
    nb = N // _R
    table = jnp.pad(pred.reshape(B * N, 3), ((0, 0), (0, 13)))
    offs = (jnp.arange(B, dtype=jnp.int32) * N)[:, None]
    idx_flat = (idx12.astype(jnp.int32) + offs).reshape(-1)
    pred_g = _sc_gather_rows(table, idx_flat)[:, :3].reshape(B, N, 3)
    gt_t = jnp.swapaxes(gt, 1, 2)
    pr_t = jnp.swapaxes(pred_g, 1, 2)

    grid = (B, nb)
    full_spec = pl.BlockSpec((1, 3, N), lambda b, j: (b, 0, 0))
    rows_spec = pl.BlockSpec((1, 3, _R), lambda b, j: (b, 0, j))
    out_spec = pl.BlockSpec((1, 1, 1, 128), lambda b, j: (b, j, 0, 0))
    partial = pl.pallas_call(
        _fused_kernel,
        grid=grid,
        in_specs=[full_spec, rows_spec, full_spec, rows_spec],
        out_specs=out_spec,
        out_shape=jax.ShapeDtypeStruct((B, nb, 1, 128), jnp.float32),
        compiler_params=pltpu.CompilerParams(
            dimension_semantics=("parallel", "parallel")),
    )(gt_t, gt_t, pr_t, pr_t)
    return jnp.sum(partial[:, :, 0, 0]) / B


# speedup vs baseline: 1.2919x; 1.2919x over previous
import functools

import jax
import jax.numpy as jnp
from jax import lax
from jax.experimental import pallas as pl
from jax.experimental.pallas import tpu as pltpu
from jax.experimental.pallas import tpu_sc as plsc

_K = 16
_EPS = 1e-8
_N = 2048
_R = 256
_SWEEPS = 6


def _jacobi3_min_eigvec(a00, a11, a22, a01, a02, a12):
    A = [[a00, a01, a02], [a01, a11, a12], [a02, a12, a22]]
    one = jnp.ones_like(a00)
    zero = jnp.zeros_like(a00)
    V = [[one, zero, zero], [zero, one, zero], [zero, zero, one]]
    for _ in range(_SWEEPS):
        for (p, q) in ((0, 2), (1, 2), (0, 1)):
            app, aqq, apq = A[p][p], A[q][q], A[p][q]
            nz = apq != 0.0
            tau = (aqq - app) / (2.0 * apq)
            sgn = jnp.where(tau >= 0.0, 1.0, -1.0)
            t = jnp.where(nz, sgn / (jnp.abs(tau) + jnp.sqrt(1.0 + tau * tau)), 0.0)
            c = jax.lax.rsqrt(1.0 + t * t)
            s = t * c
            r = 3 - p - q
            arp, arq = A[r][p], A[r][q]
            new_app = c * (c * app - s * apq) - s * (c * apq - s * aqq)
            new_aqq = s * (s * app + c * apq) + c * (s * apq + c * aqq)
            new_apq = c * s * (app - aqq) + (c * c - s * s) * apq
            new_arp = c * arp - s * arq
            new_arq = s * arp + c * arq
            A[p][p] = new_app
            A[q][q] = new_aqq
            A[p][q] = new_apq
            A[q][p] = new_apq
            A[r][p] = new_arp
            A[p][r] = new_arp
            A[r][q] = new_arq
            A[q][r] = new_arq
            for i in range(3):
                vip, viq = V[i][p], V[i][q]
                V[i][p] = c * vip - s * viq
                V[i][q] = s * vip + c * viq
    w0, w1, w2 = A[0][0], A[1][1], A[2][2]
    sel0 = (w0 <= w1) & (w0 <= w2)
    sel1 = jnp.logical_not(sel0) & (w1 <= w2)
    nx = jnp.where(sel0, V[0][0], jnp.where(sel1, V[0][1], V[0][2]))
    ny = jnp.where(sel0, V[1][0], jnp.where(sel1, V[1][1], V[1][2]))
    nz_ = jnp.where(sel0, V[2][0], jnp.where(sel1, V[2][1], V[2][2]))
    return nx, ny, nz_


def _normals_for_cloud(xt, rt):
    x, y, z = xt[0:1, :], xt[1:2, :], xt[2:3, :]
    sq_all = (x * x + y * y) + z * z
    sq_all_col = sq_all.reshape(_N, 1)
    rx, ry, rz = rt[0:1, :], rt[1:2, :], rt[2:3, :]
    sq_rows = (rx * rx + ry * ry) + rz * rz
    inner = jax.lax.dot_general(
        xt, rt, (((0,), (0,)), ((), ())),
        preferred_element_type=jnp.float32)
    dist = (sq_rows - 2.0 * inner) + sq_all_col

    iota = jax.lax.broadcasted_iota(jnp.int32, (_N, _R), 0)
    inf = jnp.float32(jnp.inf)
    for _ in range(_K):
        jmin = jnp.argmin(dist, axis=0).reshape(1, _R)
        sel = iota == jmin
        dist = jnp.where(sel, inf, dist)

    maskf = (dist == inf).astype(jnp.float32)
    mom = jax.lax.dot_general(
        xt, maskf, (((1,), (0,)), ((), ())),
        preferred_element_type=jnp.float32)
    inv_k = jnp.float32(1.0 / _K)
    mx = mom[0:1, :] * inv_k
    my = mom[1:2, :] * inv_k
    mz = mom[2:3, :] * inv_k
    xc = x.reshape(_N, 1) - mx
    yc = y.reshape(_N, 1) - my
    zc = z.reshape(_N, 1) - mz
    xm = maskf * xc
    ym = maskf * yc
    zm = maskf * zc
    ones_row = jnp.ones((1, _N), dtype=jnp.float32)

    def _colsum(v):
        return jax.lax.dot_general(
            ones_row, v, (((1,), (0,)), ((), ())),
            preferred_element_type=jnp.float32)

    cxx = _colsum(xm * xc) * inv_k
    cyy = _colsum(ym * yc) * inv_k
    czz = _colsum(zm * zc) * inv_k
    cxy = _colsum(xm * yc) * inv_k
    cxz = _colsum(xm * zc) * inv_k
    cyz = _colsum(ym * zc) * inv_k
    return _jacobi3_min_eigvec(cxx, cyy, czz, cxy, cxz, cyz)


def _fused_kernel(gt_t_ref, gt_r_ref, pr_t_ref, pr_r_ref, out_ref):
    gx, gy, gz = _normals_for_cloud(gt_t_ref[0], gt_r_ref[0])
    px, py, pz = _normals_for_cloud(pr_t_ref[0], pr_r_ref[0])
    num = gx * px + gy * py + gz * pz
    na = jnp.maximum(jnp.sqrt(gx * gx + gy * gy + gz * gz), _EPS)
    nb = jnp.maximum(jnp.sqrt(px * px + py * py + pz * pz), _EPS)
    loss = 1.0 - num / (na * nb)
    s = jnp.sum(loss)
    out_ref[...] = jnp.broadcast_to(s.reshape(1, 1, 1, 1), (1, 1, 1, 128))


def _sc_gather_rows(table, idx):
    M, D = table.shape
    B_ = idx.shape[0]
    info = plsc.get_sparse_core_info()
    nw = info.num_cores * info.num_subcores
    b_per_w = B_ // nw
    mesh = plsc.VectorSubcoreMesh(core_axis_name="c", subcore_axis_name="s")

    @functools.partial(
        pl.kernel, mesh=mesh,
        out_type=jax.ShapeDtypeStruct((B_, D), jnp.float32),
        scratch_types=[
            pltpu.VMEM((b_per_w,), jnp.int32),
            pltpu.VMEM((b_per_w, D), jnp.float32),
            pltpu.SemaphoreType.DMA,
        ],
        compiler_params=pltpu.CompilerParams(use_tc_tiling_on_sc=False),
    )
    def gather_kernel(table_hbm, idx_hbm, out_hbm, idx_v, rows_v, sem):
        wid = lax.axis_index("s") * info.num_cores + lax.axis_index("c")
        base = wid * b_per_w
        pltpu.sync_copy(idx_hbm.at[pl.ds(base, b_per_w)], idx_v)
        pltpu.async_copy(table_hbm.at[idx_v], rows_v, sem).wait()
        pltpu.sync_copy(rows_v, out_hbm.at[pl.ds(base, b_per_w)])

    return gather_kernel(table, idx)


@jax.jit
def kernel(gt, pred, idx12):
    B, N, _ = gt.shape
    nb = N // _R
    table = jnp.pad(pred.reshape(B * N, 3), ((0, 0), (0, 13)))
    offs = (jnp.arange(B, dtype=jnp.int32) * N)[:, None]
    idx_flat = (idx12.astype(jnp.int32) + offs).reshape(-1)
    pred_g = _sc_gather_rows(table, idx_flat)[:, :3].reshape(B, N, 3)
    gt_t = jnp.swapaxes(gt, 1, 2)
    pr_t = jnp.swapaxes(pred_g, 1, 2)

    grid = (B, nb)
    full_spec = pl.BlockSpec((1, 3, N), lambda b, j: (b, 0, 0))
    rows_spec = pl.BlockSpec((1, 3, _R), lambda b, j: (b, 0, j))
    out_spec = pl.BlockSpec((1, 1, 1, 128), lambda b, j: (b, j, 0, 0))
    partial = pl.pallas_call(
        _fused_kernel,
        grid=grid,
        in_specs=[full_spec, rows_spec, full_spec, rows_spec],
        out_specs=out_spec,
        out_shape=jax.ShapeDtypeStruct((B, nb, 1, 128), jnp.float32),
        compiler_params=pltpu.CompilerParams(
            dimension_semantics=("arbitrary", "arbitrary")),
    )(gt_t, gt_t, pr_t, pr_t)
    return jnp.sum(partial[:, :, 0, 0]) / B


# speedup vs baseline: 1.2956x; 1.0028x over previous
import functools

import jax
import jax.numpy as jnp
from jax import lax
from jax.experimental import pallas as pl
from jax.experimental.pallas import tpu as pltpu
from jax.experimental.pallas import tpu_sc as plsc

_K = 16
_EPS = 1e-8
_N = 2048
_R = 256
_SWEEPS = 5


def _jacobi3_min_eigvec(a00, a11, a22, a01, a02, a12):
    A = [[a00, a01, a02], [a01, a11, a12], [a02, a12, a22]]
    one = jnp.ones_like(a00)
    zero = jnp.zeros_like(a00)
    V = [[one, zero, zero], [zero, one, zero], [zero, zero, one]]
    for _ in range(_SWEEPS):
        for (p, q) in ((0, 2), (1, 2), (0, 1)):
            app, aqq, apq = A[p][p], A[q][q], A[p][q]
            nz = apq != 0.0
            tau = (aqq - app) / (2.0 * apq)
            sgn = jnp.where(tau >= 0.0, 1.0, -1.0)
            t = jnp.where(nz, sgn / (jnp.abs(tau) + jnp.sqrt(1.0 + tau * tau)), 0.0)
            c = jax.lax.rsqrt(1.0 + t * t)
            s = t * c
            r = 3 - p - q
            arp, arq = A[r][p], A[r][q]
            new_app = c * (c * app - s * apq) - s * (c * apq - s * aqq)
            new_aqq = s * (s * app + c * apq) + c * (s * apq + c * aqq)
            new_apq = c * s * (app - aqq) + (c * c - s * s) * apq
            new_arp = c * arp - s * arq
            new_arq = s * arp + c * arq
            A[p][p] = new_app
            A[q][q] = new_aqq
            A[p][q] = new_apq
            A[q][p] = new_apq
            A[r][p] = new_arp
            A[p][r] = new_arp
            A[r][q] = new_arq
            A[q][r] = new_arq
            for i in range(3):
                vip, viq = V[i][p], V[i][q]
                V[i][p] = c * vip - s * viq
                V[i][q] = s * vip + c * viq
    w0, w1, w2 = A[0][0], A[1][1], A[2][2]
    sel0 = (w0 <= w1) & (w0 <= w2)
    sel1 = jnp.logical_not(sel0) & (w1 <= w2)
    nx = jnp.where(sel0, V[0][0], jnp.where(sel1, V[0][1], V[0][2]))
    ny = jnp.where(sel0, V[1][0], jnp.where(sel1, V[1][1], V[1][2]))
    nz_ = jnp.where(sel0, V[2][0], jnp.where(sel1, V[2][1], V[2][2]))
    return nx, ny, nz_


def _normals_for_cloud(xt, rt):
    x, y, z = xt[0:1, :], xt[1:2, :], xt[2:3, :]
    sq_all = (x * x + y * y) + z * z
    sq_all_col = sq_all.reshape(_N, 1)
    rx, ry, rz = rt[0:1, :], rt[1:2, :], rt[2:3, :]
    sq_rows = (rx * rx + ry * ry) + rz * rz
    inner = jax.lax.dot_general(
        xt, rt, (((0,), (0,)), ((), ())),
        preferred_element_type=jnp.float32)
    dist = (sq_rows - 2.0 * inner) + sq_all_col

    iota = jax.lax.broadcasted_iota(jnp.int32, (_N, _R), 0)
    inf = jnp.float32(jnp.inf)
    for _ in range(_K - 1):
        jmin = jnp.argmin(dist, axis=0).reshape(1, _R)
        dist = jnp.where(iota == jmin, inf, dist)
    jmin = jnp.argmin(dist, axis=0).reshape(1, _R)
    maskf = ((dist == inf) | (iota == jmin)).astype(jnp.float32)
    mom = jax.lax.dot_general(
        xt, maskf, (((1,), (0,)), ((), ())),
        preferred_element_type=jnp.float32)
    inv_k = jnp.float32(1.0 / _K)
    mx = mom[0:1, :] * inv_k
    my = mom[1:2, :] * inv_k
    mz = mom[2:3, :] * inv_k
    xc = x.reshape(_N, 1) - mx
    yc = y.reshape(_N, 1) - my
    zc = z.reshape(_N, 1) - mz
    xm = maskf * xc
    ym = maskf * yc
    zm = maskf * zc
    ones_row = jnp.ones((1, _N), dtype=jnp.float32)

    def _colsum(v):
        return jax.lax.dot_general(
            ones_row, v, (((1,), (0,)), ((), ())),
            preferred_element_type=jnp.float32)

    cxx = _colsum(xm * xc) * inv_k
    cyy = _colsum(ym * yc) * inv_k
    czz = _colsum(zm * zc) * inv_k
    cxy = _colsum(xm * yc) * inv_k
    cxz = _colsum(xm * zc) * inv_k
    cyz = _colsum(ym * zc) * inv_k
    return _jacobi3_min_eigvec(cxx, cyy, czz, cxy, cxz, cyz)


def _fused_kernel(gt_t_ref, gt_r_ref, pr_t_ref, pr_r_ref, out_ref):
    gx, gy, gz = _normals_for_cloud(gt_t_ref[0], gt_r_ref[0])
    px, py, pz = _normals_for_cloud(pr_t_ref[0], pr_r_ref[0])
    num = gx * px + gy * py + gz * pz
    na = jnp.maximum(jnp.sqrt(gx * gx + gy * gy + gz * gz), _EPS)
    nb = jnp.maximum(jnp.sqrt(px * px + py * py + pz * pz), _EPS)
    loss = 1.0 - num / (na * nb)
    s = jnp.sum(loss)
    out_ref[...] = jnp.broadcast_to(s.reshape(1, 1, 1, 1), (1, 1, 1, 128))


def _sc_gather_rows(table, idx):
    M, D = table.shape
    B_ = idx.shape[0]
    info = plsc.get_sparse_core_info()
    nw = info.num_cores * info.num_subcores
    b_per_w = B_ // nw
    mesh = plsc.VectorSubcoreMesh(core_axis_name="c", subcore_axis_name="s")

    @functools.partial(
        pl.kernel, mesh=mesh,
        out_type=jax.ShapeDtypeStruct((B_, D), jnp.float32),
        scratch_types=[
            pltpu.VMEM((b_per_w,), jnp.int32),
            pltpu.VMEM((b_per_w, D), jnp.float32),
            pltpu.SemaphoreType.DMA,
        ],
        compiler_params=pltpu.CompilerParams(use_tc_tiling_on_sc=False),
    )
    def gather_kernel(table_hbm, idx_hbm, out_hbm, idx_v, rows_v, sem):
        wid = lax.axis_index("s") * info.num_cores + lax.axis_index("c")
        base = wid * b_per_w
        pltpu.sync_copy(idx_hbm.at[pl.ds(base, b_per_w)], idx_v)
        pltpu.async_copy(table_hbm.at[idx_v], rows_v, sem).wait()
        pltpu.sync_copy(rows_v, out_hbm.at[pl.ds(base, b_per_w)])

    return gather_kernel(table, idx)


@jax.jit
def kernel(gt, pred, idx12):
    B, N, _ = gt.shape
    nb = N // _R
    table = jnp.pad(pred.reshape(B * N, 3), ((0, 0), (0, 13)))
    offs = (jnp.arange(B, dtype=jnp.int32) * N)[:, None]
    idx_flat = (idx12.astype(jnp.int32) + offs).reshape(-1)
    pred_g = _sc_gather_rows(table, idx_flat)[:, :3].reshape(B, N, 3)
    gt_t = jnp.swapaxes(gt, 1, 2)
    pr_t = jnp.swapaxes(pred_g, 1, 2)

    grid = (B, nb)
    full_spec = pl.BlockSpec((1, 3, N), lambda b, j: (b, 0, 0))
    rows_spec = pl.BlockSpec((1, 3, _R), lambda b, j: (b, 0, j))
    out_spec = pl.BlockSpec((1, 1, 1, 128), lambda b, j: (b, j, 0, 0))
    partial = pl.pallas_call(
        _fused_kernel,
        grid=grid,
        in_specs=[full_spec, rows_spec, full_spec, rows_spec],
        out_specs=out_spec,
        out_shape=jax.ShapeDtypeStruct((B, nb, 1, 128), jnp.float32),
        compiler_params=pltpu.CompilerParams(
            dimension_semantics=("arbitrary", "arbitrary")),
    )(gt_t, gt_t, pr_t, pr_t)
    return jnp.sum(partial[:, :, 0, 0]) / B
